# baseline (speedup 1.0000x reference)
import functools

import jax
import jax.numpy as jnp
from jax import lax
from jax.experimental import pallas as pl
from jax.experimental.pallas import tpu as pltpu
from jax.experimental.pallas import tpu_sc as plsc

N_CODES = 8192
DIM = 32
TOK_TILE = 8192
CODE_CHUNK = 1024
GATHER_CHUNK = 128
LANES = 128


def _argmin_body(z_ref, embs_ref, idx_ref, e2_ref):
    n_chunks = N_CODES // CODE_CHUNK
    n_slices = CODE_CHUNK // LANES

    @pl.when(pl.program_id(0) == 0)
    def _():
        e = embs_ref[...]
        e2_ref[...] = jnp.sum(e * e, axis=-1).reshape(N_CODES // LANES, LANES)

    z = z_ref[...]
    z2 = jnp.sum(z * z, axis=-1, keepdims=True)
    zz = z + z

    m_run = jnp.full((TOK_TILE, LANES), jnp.inf, jnp.float32)
    s_run = jnp.zeros((TOK_TILE, LANES), jnp.int32)
    for c in range(n_chunks):
        e = embs_ref[pl.ds(c * CODE_CHUNK, CODE_CHUNK), :]
        sim2 = lax.dot_general(zz, e, (((1,), (1,)), ((), ())))
        for s in range(n_slices):
            sim2_s = lax.slice(sim2, (0, s * LANES), (TOK_TILE, (s + 1) * LANES))
            e2_s = e2_ref[pl.ds(c * n_slices + s, 1), :]
            dist = (z2 + e2_s) - sim2_s
            better = dist < m_run
            m_run = jnp.minimum(dist, m_run)
            s_run = jnp.where(better, c * n_slices + s, s_run)

    m_fin = jnp.min(m_run, axis=1, keepdims=True)
    lane = lax.broadcasted_iota(jnp.int32, (TOK_TILE, LANES), 1)
    full_idx = s_run * LANES + lane
    cand = jnp.where(m_run == m_fin, full_idx, 2**30)
    idx = jnp.min(cand, axis=1)
    idx_ref[...] = idx.reshape(idx_ref.shape)


def _code_indices(z, embs):
    n_tok = z.shape[0]
    grid = n_tok // TOK_TILE
    idx3 = pl.pallas_call(
        _argmin_body,
        grid=(grid,),
        in_specs=[
            pl.BlockSpec((TOK_TILE, DIM), lambda i: (i, 0)),
            pl.BlockSpec((N_CODES, DIM), lambda i: (0, 0)),
        ],
        out_specs=pl.BlockSpec((1, 1, TOK_TILE), lambda i: (i, 0, 0)),
        out_shape=jax.ShapeDtypeStruct((grid, 1, TOK_TILE), jnp.int32),
        scratch_shapes=[pltpu.VMEM((N_CODES // LANES, LANES), jnp.float32)],
        compiler_params=pltpu.CompilerParams(
            dimension_semantics=("arbitrary",)),
    )(z, embs)
    return idx3.reshape(n_tok)


def _gather_rows(embs, idx):
    n_tok = idx.shape[0]
    mesh = plsc.VectorSubcoreMesh(core_axis_name="c", subcore_axis_name="s")
    info = plsc.get_sparse_core_info()
    n_workers = info.num_cores * info.num_subcores
    per_w = n_tok // n_workers
    n_sub = per_w // GATHER_CHUNK
    idx2d = idx.reshape(n_tok // GATHER_CHUNK, GATHER_CHUNK)

    @functools.partial(
        pl.kernel, mesh=mesh,
        out_type=jax.ShapeDtypeStruct((n_tok, DIM), jnp.float32),
        scratch_types=[
            pltpu.VMEM((n_sub, GATHER_CHUNK), jnp.int32),
            pltpu.VMEM((per_w, DIM), jnp.float32),
            pltpu.SemaphoreType.DMA,
        ],
        compiler_params=pltpu.CompilerParams(use_tc_tiling_on_sc=False),
    )
    def gather_k(embs_hbm, idx_hbm, out_hbm, idx_v, rows_v, sem):
        wid = lax.axis_index("s") * info.num_cores + lax.axis_index("c")
        base = wid * per_w
        pltpu.sync_copy(idx_hbm.at[pl.ds(wid * n_sub, n_sub), :], idx_v)
        copies = [
            pltpu.async_copy(
                embs_hbm.at[idx_v.at[j]],
                rows_v.at[pl.ds(j * GATHER_CHUNK, GATHER_CHUNK), :],
                sem,
            )
            for j in range(n_sub)
        ]
        for cp in copies:
            cp.wait()
        pltpu.sync_copy(rows_v, out_hbm.at[pl.ds(base, per_w)])

    return gather_k(embs, idx2d)


def kernel(ze, embs):
    b, h, w, c = ze.shape
    z = ze.reshape(-1, c)
    idx = _code_indices(z, embs)
    zq = _gather_rows(embs, idx)
    return zq.reshape(b, h, w, c)


# speedup vs baseline: 1.2042x; 1.2042x over previous
import functools

import jax
import jax.numpy as jnp
from jax import lax
from jax.experimental import pallas as pl
from jax.experimental.pallas import tpu as pltpu
from jax.experimental.pallas import tpu_sc as plsc

N_CODES = 8192
DIM = 32
TOK_TILE = 2048
CODE_CHUNK = 4096
GATHER_CHUNK = 128
LANES = 128


def _argmin_body(z_ref, embs_ref, idx_ref, e2_ref):
    n_chunks = N_CODES // CODE_CHUNK
    n_slices = CODE_CHUNK // LANES

    @pl.when(pl.program_id(0) == 0)
    def _():
        e = embs_ref[...]
        e2_ref[...] = jnp.sum(e * e, axis=-1).reshape(N_CODES // LANES, LANES)

    z = z_ref[...]
    z2 = jnp.sum(z * z, axis=-1, keepdims=True)
    zz = z + z

    m_run = jnp.full((TOK_TILE, LANES), jnp.inf, jnp.float32)
    s_run = jnp.zeros((TOK_TILE, LANES), jnp.int32)
    for c in range(n_chunks):
        e = embs_ref[pl.ds(c * CODE_CHUNK, CODE_CHUNK), :]
        sim2 = lax.dot_general(zz, e, (((1,), (1,)), ((), ())))
        for s in range(n_slices):
            sim2_s = lax.slice(sim2, (0, s * LANES), (TOK_TILE, (s + 1) * LANES))
            e2_s = e2_ref[pl.ds(c * n_slices + s, 1), :]
            dist = (z2 + e2_s) - sim2_s
            better = dist < m_run
            m_run = jnp.minimum(dist, m_run)
            s_run = jnp.where(better, c * n_slices + s, s_run)

    m_fin = jnp.min(m_run, axis=1, keepdims=True)
    lane = lax.broadcasted_iota(jnp.int32, (TOK_TILE, LANES), 1)
    full_idx = s_run * LANES + lane
    cand = jnp.where(m_run == m_fin, full_idx, 2**30)
    idx = jnp.min(cand, axis=1)
    idx_ref[...] = idx.reshape(idx_ref.shape)


def _code_indices(z, embs):
    n_tok = z.shape[0]
    grid = n_tok // TOK_TILE
    idx3 = pl.pallas_call(
        _argmin_body,
        grid=(grid,),
        in_specs=[
            pl.BlockSpec((TOK_TILE, DIM), lambda i: (i, 0)),
            pl.BlockSpec((N_CODES, DIM), lambda i: (0, 0)),
        ],
        out_specs=pl.BlockSpec((1, 1, TOK_TILE), lambda i: (i, 0, 0)),
        out_shape=jax.ShapeDtypeStruct((grid, 1, TOK_TILE), jnp.int32),
        scratch_shapes=[pltpu.VMEM((N_CODES // LANES, LANES), jnp.float32)],
        compiler_params=pltpu.CompilerParams(
            dimension_semantics=("arbitrary",)),
    )(z, embs)
    return idx3.reshape(n_tok)


def _gather_rows(embs, idx):
    n_tok = idx.shape[0]
    mesh = plsc.VectorSubcoreMesh(core_axis_name="c", subcore_axis_name="s")
    info = plsc.get_sparse_core_info()
    n_workers = info.num_cores * info.num_subcores
    per_w = n_tok // n_workers
    n_sub = per_w // GATHER_CHUNK
    idx2d = idx.reshape(n_tok // GATHER_CHUNK, GATHER_CHUNK)

    @functools.partial(
        pl.kernel, mesh=mesh,
        out_type=jax.ShapeDtypeStruct((n_tok, DIM), jnp.float32),
        scratch_types=[
            pltpu.VMEM((n_sub, GATHER_CHUNK), jnp.int32),
            pltpu.VMEM((per_w, DIM), jnp.float32),
            pltpu.SemaphoreType.DMA,
        ],
        compiler_params=pltpu.CompilerParams(use_tc_tiling_on_sc=False),
    )
    def gather_k(embs_hbm, idx_hbm, out_hbm, idx_v, rows_v, sem):
        wid = lax.axis_index("s") * info.num_cores + lax.axis_index("c")
        base = wid * per_w
        pltpu.sync_copy(idx_hbm.at[pl.ds(wid * n_sub, n_sub), :], idx_v)
        copies = [
            pltpu.async_copy(
                embs_hbm.at[idx_v.at[j]],
                rows_v.at[pl.ds(j * GATHER_CHUNK, GATHER_CHUNK), :],
                sem,
            )
            for j in range(n_sub)
        ]
        for cp in copies:
            cp.wait()
        pltpu.sync_copy(rows_v, out_hbm.at[pl.ds(base, per_w)])

    return gather_k(embs, idx2d)


def kernel(ze, embs):
    b, h, w, c = ze.shape
    z = ze.reshape(-1, c)
    idx = _code_indices(z, embs)
    zq = _gather_rows(embs, idx)
    return zq.reshape(b, h, w, c)


# speedup vs baseline: 1.2344x; 1.0251x over previous
import functools

import jax
import jax.numpy as jnp
from jax import lax
from jax.experimental import pallas as pl
from jax.experimental.pallas import tpu as pltpu
from jax.experimental.pallas import tpu_sc as plsc

N_CODES = 8192
DIM = 32
TOK_TILE = 4096
CODE_CHUNK = 2048
GATHER_CHUNK = 128
LANES = 128


def _argmin_body(z_ref, embs_ref, idx_ref, e2_ref):
    n_chunks = N_CODES // CODE_CHUNK
    n_slices = CODE_CHUNK // LANES

    @pl.when(pl.program_id(0) == 0)
    def _():
        e = embs_ref[...]
        e2_ref[...] = jnp.sum(e * e, axis=-1).reshape(N_CODES // LANES, LANES)

    z = z_ref[...]
    z2 = jnp.sum(z * z, axis=-1, keepdims=True)
    zz = z + z

    m_run = jnp.full((TOK_TILE, LANES), jnp.inf, jnp.float32)
    s_run = jnp.zeros((TOK_TILE, LANES), jnp.int32)
    for c in range(n_chunks):
        e = embs_ref[pl.ds(c * CODE_CHUNK, CODE_CHUNK), :]
        sim2 = lax.dot_general(zz, e, (((1,), (1,)), ((), ())))
        for s in range(n_slices):
            sim2_s = lax.slice(sim2, (0, s * LANES), (TOK_TILE, (s + 1) * LANES))
            e2_s = e2_ref[pl.ds(c * n_slices + s, 1), :]
            dist = (z2 + e2_s) - sim2_s
            better = dist < m_run
            m_run = jnp.minimum(dist, m_run)
            s_run = jnp.where(better, c * n_slices + s, s_run)

    m_fin = jnp.min(m_run, axis=1, keepdims=True)
    lane = lax.broadcasted_iota(jnp.int32, (TOK_TILE, LANES), 1)
    full_idx = s_run * LANES + lane
    cand = jnp.where(m_run == m_fin, full_idx, 2**30)
    idx = jnp.min(cand, axis=1)
    idx_ref[...] = idx.reshape(idx_ref.shape)


def _code_indices(z, embs):
    n_tok = z.shape[0]
    grid = n_tok // TOK_TILE
    idx3 = pl.pallas_call(
        _argmin_body,
        grid=(grid,),
        in_specs=[
            pl.BlockSpec((TOK_TILE, DIM), lambda i: (i, 0)),
            pl.BlockSpec((N_CODES, DIM), lambda i: (0, 0)),
        ],
        out_specs=pl.BlockSpec((1, 1, TOK_TILE), lambda i: (i, 0, 0)),
        out_shape=jax.ShapeDtypeStruct((grid, 1, TOK_TILE), jnp.int32),
        scratch_shapes=[pltpu.VMEM((N_CODES // LANES, LANES), jnp.float32)],
        compiler_params=pltpu.CompilerParams(
            dimension_semantics=("parallel",)),
    )(z, embs)
    return idx3.reshape(n_tok)


def _gather_rows(embs, idx):
    n_tok = idx.shape[0]
    mesh = plsc.VectorSubcoreMesh(core_axis_name="c", subcore_axis_name="s")
    info = plsc.get_sparse_core_info()
    n_workers = info.num_cores * info.num_subcores
    per_w = n_tok // n_workers
    n_sub = per_w // GATHER_CHUNK
    idx2d = idx.reshape(n_tok // GATHER_CHUNK, GATHER_CHUNK)

    @functools.partial(
        pl.kernel, mesh=mesh,
        out_type=jax.ShapeDtypeStruct((n_tok, DIM), jnp.float32),
        scratch_types=[
            pltpu.VMEM((n_sub, GATHER_CHUNK), jnp.int32),
            pltpu.VMEM((per_w, DIM), jnp.float32),
            pltpu.SemaphoreType.DMA,
        ],
        compiler_params=pltpu.CompilerParams(use_tc_tiling_on_sc=False),
    )
    def gather_k(embs_hbm, idx_hbm, out_hbm, idx_v, rows_v, sem):
        wid = lax.axis_index("s") * info.num_cores + lax.axis_index("c")
        base = wid * per_w
        pltpu.sync_copy(idx_hbm.at[pl.ds(wid * n_sub, n_sub), :], idx_v)
        copies = [
            pltpu.async_copy(
                embs_hbm.at[idx_v.at[j]],
                rows_v.at[pl.ds(j * GATHER_CHUNK, GATHER_CHUNK), :],
                sem,
            )
            for j in range(n_sub)
        ]
        for cp in copies:
            cp.wait()
        pltpu.sync_copy(rows_v, out_hbm.at[pl.ds(base, per_w)])

    return gather_k(embs, idx2d)


def kernel(ze, embs):
    b, h, w, c = ze.shape
    z = ze.reshape(-1, c)
    idx = _code_indices(z, embs)
    zq = _gather_rows(embs, idx)
    return zq.reshape(b, h, w, c)
